# baseline (speedup 1.0000x reference)
import functools
import math

import jax
import jax.numpy as jnp
from jax import lax
from jax.experimental import pallas as pl
from jax.experimental.pallas import tpu as pltpu
from jax.experimental.pallas import tpu_sc as plsc

VOCAB = 100000
D = 1024
S = 8192
B = 4

NC = 2
NS = 16
NW = NC * NS
S_PER_W = S // NW
C = 8
NCH = S_PER_W // C
NT = NCH * B
NR = 8

NH = S // 64
NL = 64

_LN10000 = math.log(10000.0)


_HPB = 16
_PBS = _HPB * NL


def _inv_denom_cols(shape, dim):
    col = lax.broadcasted_iota(jnp.int32, shape, dim)
    i_val = ((col >> 1) << 1).astype(jnp.float32)
    return jnp.exp(i_val * (_LN10000 / float(D)))


def _pos_body(out_ref, sinl_ref, cosl_ref, a_ref, b_ref):
    g = pl.program_id(0)

    @pl.when(g == 0)
    def _():
        al = lax.broadcasted_iota(jnp.int32, (NL, D), 0).astype(
            jnp.float32
        ) / _inv_denom_cols((NL, D), 1)
        sinl_ref[...] = jnp.sin(al)
        cosl_ref[...] = jnp.cos(al)
        ah = (
            lax.broadcasted_iota(jnp.int32, (NH, D), 0) * 64
        ).astype(jnp.float32) / _inv_denom_cols((NH, D), 1)
        sh = jnp.sin(ah)
        ch_ = jnp.cos(ah)
        col = lax.broadcasted_iota(jnp.int32, (NH, D), 1)
        even = (col & 1) == 0
        a_ref[...] = jnp.where(even, sh, ch_)
        b_ref[...] = jnp.where(even, ch_, -sh)

    hs = pl.ds(g * _HPB, _HPB)
    a = a_ref[hs, :][:, None, :]
    b = b_ref[hs, :][:, None, :]
    sl = sinl_ref[...][None, :, :]
    cl = cosl_ref[...][None, :, :]
    res = a * cl + b * sl
    out_ref[...] = res.reshape(_PBS, D)


def _make_pos():
    return pl.pallas_call(
        _pos_body,
        out_shape=jax.ShapeDtypeStruct((S, D), jnp.float32),
        grid=(S // _PBS,),
        out_specs=pl.BlockSpec((_PBS, D), lambda g: (g, 0)),
        scratch_shapes=[
            pltpu.VMEM((NL, D), jnp.float32),
            pltpu.VMEM((NL, D), jnp.float32),
            pltpu.VMEM((NH, D), jnp.float32),
            pltpu.VMEM((NH, D), jnp.float32),
        ],
    )()


def _add_pos(rows_v, pos_v):
    def add_row(r, _):
        def add_slice(j, _):
            off = j * 16
            rows_v[r, pl.ds(off, 16)] = (
                rows_v[r, pl.ds(off, 16)] + pos_v[r, pl.ds(off, 16)]
            )
            return 0

        lax.fori_loop(0, D // 16, add_slice, 0, unroll=4)
        return 0

    lax.fori_loop(0, C, add_row, 0)


def _sc_body(pos_hbm, x_hbm, tab_hbm, out_hbm, *scr):
    cid = lax.axis_index("c")
    sid = lax.axis_index("s")
    wid = sid * NC + cid
    w0 = wid * S_PER_W

    idx_v = scr[0]
    pos = scr[1:3]
    rows = scr[3 : 3 + NR]
    gsem = scr[3 + NR : 3 + 2 * NR]
    ssem = scr[3 + 2 * NR : 3 + 3 * NR]
    psem = scr[3 + 3 * NR : 3 + 3 * NR + 2]

    def gather_start(ch, b, buf):
        idx = idx_v.at[b, pl.ds(pl.multiple_of(ch * C, C), C)]
        pltpu.make_async_copy(tab_hbm.at[idx], rows[buf], gsem[buf]).start()

    def gather_wait(buf):
        idx = idx_v.at[0, pl.ds(0, C)]
        pltpu.make_async_copy(tab_hbm.at[idx], rows[buf], gsem[buf]).wait()

    def store_start(ch, b, buf):
        base = pl.multiple_of(b * S + w0 + ch * C, C)
        pltpu.make_async_copy(
            rows[buf], out_hbm.at[pl.ds(base, C), :], ssem[buf]
        ).start()

    def store_wait(buf):
        pltpu.make_async_copy(
            rows[buf], out_hbm.at[pl.ds(w0, C), :], ssem[buf]
        ).wait()

    def pos_start(ch, pbuf):
        src = pos_hbm.at[pl.ds(pl.multiple_of(w0 + ch * C, C), C), :]
        pltpu.make_async_copy(src, pos[pbuf], psem[pbuf]).start()

    def pos_wait(pbuf):
        src = pos_hbm.at[pl.ds(pl.multiple_of(w0, C), C), :]
        pltpu.make_async_copy(src, pos[pbuf], psem[pbuf]).wait()

    for b in range(B):
        pltpu.sync_copy(
            x_hbm.at[pl.ds(pl.multiple_of(b * S + w0, C), S_PER_W)],
            idx_v.at[b],
        )
    pos_start(0, 0)
    pos_start(1, 1)
    gather_start(0, 0, 0)

    def body(i, _):
        for half in range(2):
            cc = i * 2 + half
            pbuf = half
            for k in range(B):
                buf = half * B + k
                if half == 1 and k == B - 1:
                    store_wait((buf + 1) % NR)
                else:

                    @pl.when(i > 0)
                    def _():
                        store_wait((buf + 1) % NR)

                if k < B - 1:
                    gather_start(cc, k + 1, buf + 1)
                elif half == 0:
                    gather_start(cc + 1, 0, (buf + 1) % NR)
                else:

                    @pl.when(cc < NCH - 1)
                    def _():
                        gather_start(cc + 1, 0, (buf + 1) % NR)

                if k == 0:
                    pos_wait(pbuf)
                gather_wait(buf)
                _add_pos(rows[buf], pos[pbuf])
                store_start(cc, k, buf)
                if k == B - 1:

                    @pl.when(cc + 2 < NCH)
                    def _():
                        pos_start(cc + 2, pbuf)

        return 0

    lax.fori_loop(0, NCH // 2, body, 0)
    for b in range(1, NR):
        store_wait(b)


def _make_sc():
    mesh = plsc.VectorSubcoreMesh(core_axis_name="c", subcore_axis_name="s")
    f32 = jnp.float32
    return functools.partial(
        pl.kernel,
        mesh=mesh,
        out_type=jax.ShapeDtypeStruct((B * S, D), f32),
        scratch_types=[pltpu.VMEM((B, S_PER_W), jnp.int32)]
        + [pltpu.VMEM((C, D), f32)] * (2 + NR)
        + [pltpu.SemaphoreType.DMA] * (2 * NR + 2),
    )(_sc_body)


@jax.jit
def kernel(x, tok_table):
    pos = _make_pos()
    out = _make_sc()(pos, x.reshape(B * S), tok_table)
    return out.reshape(B, S, D)


# speedup vs baseline: 2.5282x; 2.5282x over previous
import functools
import math

import jax
import jax.numpy as jnp
from jax import lax
from jax.experimental import pallas as pl
from jax.experimental.pallas import tpu as pltpu
from jax.experimental.pallas import tpu_sc as plsc

VOCAB = 100000
D = 1024
S = 8192
B = 4

NC = 2
NS = 16
NW = NC * NS
S_PER_W = S // NW
C = 16
NCH = S_PER_W // C
NT = NCH * B
NR = 4

NH = S // 64
NL = 64

_LN10000 = math.log(10000.0)


_HPB = 16
_PBS = _HPB * NL


def _inv_denom_cols(shape, dim):
    col = lax.broadcasted_iota(jnp.int32, shape, dim)
    i_val = ((col >> 1) << 1).astype(jnp.float32)
    return jnp.exp(i_val * (_LN10000 / float(D)))


def _pos_body(out_ref, sinl_ref, cosl_ref, a_ref, b_ref):
    g = pl.program_id(0)

    @pl.when(g == 0)
    def _():
        al = lax.broadcasted_iota(jnp.int32, (NL, D), 0).astype(
            jnp.float32
        ) / _inv_denom_cols((NL, D), 1)
        sinl_ref[...] = jnp.sin(al)
        cosl_ref[...] = jnp.cos(al)
        ah = (
            lax.broadcasted_iota(jnp.int32, (NH, D), 0) * 64
        ).astype(jnp.float32) / _inv_denom_cols((NH, D), 1)
        sh = jnp.sin(ah)
        ch_ = jnp.cos(ah)
        col = lax.broadcasted_iota(jnp.int32, (NH, D), 1)
        even = (col & 1) == 0
        a_ref[...] = jnp.where(even, sh, ch_)
        b_ref[...] = jnp.where(even, ch_, -sh)

    hs = pl.ds(g * _HPB, _HPB)
    a = a_ref[hs, :][:, None, :]
    b = b_ref[hs, :][:, None, :]
    sl = sinl_ref[...][None, :, :]
    cl = cosl_ref[...][None, :, :]
    res = a * cl + b * sl
    out_ref[...] = res.reshape(_PBS, D)


def _make_pos():
    return pl.pallas_call(
        _pos_body,
        out_shape=jax.ShapeDtypeStruct((S, D), jnp.float32),
        grid=(S // _PBS,),
        out_specs=pl.BlockSpec((_PBS, D), lambda g: (g, 0)),
        scratch_shapes=[
            pltpu.VMEM((NL, D), jnp.float32),
            pltpu.VMEM((NL, D), jnp.float32),
            pltpu.VMEM((NH, D), jnp.float32),
            pltpu.VMEM((NH, D), jnp.float32),
        ],
    )()


def _add_pos(rows_v, pos_v):
    def add_row(r, _):
        def add_slice(j, _):
            off = j * 16
            rows_v[r, pl.ds(off, 16)] = (
                rows_v[r, pl.ds(off, 16)] + pos_v[r, pl.ds(off, 16)]
            )
            return 0

        lax.fori_loop(0, D // 16, add_slice, 0, unroll=4)
        return 0

    lax.fori_loop(0, C, add_row, 0)


def _sc_body(pos_hbm, x_hbm, tab_hbm, out_hbm, *scr):
    cid = lax.axis_index("c")
    sid = lax.axis_index("s")
    wid = sid * NC + cid
    w0 = wid * S_PER_W

    idx_v = scr[0]
    pos = scr[1:3]
    rows = scr[3 : 3 + NR]
    gsem = scr[3 + NR : 3 + 2 * NR]
    ssem = scr[3 + 2 * NR : 3 + 3 * NR]
    psem = scr[3 + 3 * NR : 3 + 3 * NR + 2]

    def gather_start(ch, b, buf):
        idx = idx_v.at[b, pl.ds(pl.multiple_of(ch * C, C), C)]
        pltpu.make_async_copy(tab_hbm.at[idx], rows[buf], gsem[buf]).start()

    def gather_wait(buf):
        idx = idx_v.at[0, pl.ds(0, C)]
        pltpu.make_async_copy(tab_hbm.at[idx], rows[buf], gsem[buf]).wait()

    def store_start(ch, b, buf):
        base = pl.multiple_of(b * S + w0 + ch * C, C)
        pltpu.make_async_copy(
            rows[buf], out_hbm.at[pl.ds(base, C), :], ssem[buf]
        ).start()

    def store_wait(buf):
        pltpu.make_async_copy(
            rows[buf], out_hbm.at[pl.ds(w0, C), :], ssem[buf]
        ).wait()

    def pos_start(ch, pbuf):
        src = pos_hbm.at[pl.ds(pl.multiple_of(w0 + ch * C, C), C), :]
        pltpu.make_async_copy(src, pos[pbuf], psem[pbuf]).start()

    def pos_wait(pbuf):
        src = pos_hbm.at[pl.ds(pl.multiple_of(w0, C), C), :]
        pltpu.make_async_copy(src, pos[pbuf], psem[pbuf]).wait()

    for b in range(B):
        pltpu.sync_copy(
            x_hbm.at[pl.ds(pl.multiple_of(b * S + w0, C), S_PER_W)],
            idx_v.at[b],
        )
    pos_start(0, 0)
    pos_start(1, 1)
    gather_start(0, 0, 0)

    def body(i, _):
        for half in range(2):
            cc = i * 2 + half
            pbuf = half
            for k in range(B):
                buf = (half * B + k) % NR
                if half * B + k >= NR - 1:
                    store_wait((buf + 1) % NR)
                else:

                    @pl.when(i > 0)
                    def _():
                        store_wait((buf + 1) % NR)

                if k < B - 1:
                    gather_start(cc, k + 1, buf + 1)
                elif half == 0:
                    gather_start(cc + 1, 0, (buf + 1) % NR)
                else:

                    @pl.when(cc < NCH - 1)
                    def _():
                        gather_start(cc + 1, 0, (buf + 1) % NR)

                if k == 0:
                    pos_wait(pbuf)
                gather_wait(buf)
                _add_pos(rows[buf], pos[pbuf])
                store_start(cc, k, buf)
                if k == B - 1:

                    @pl.when(cc + 2 < NCH)
                    def _():
                        pos_start(cc + 2, pbuf)

        return 0

    lax.fori_loop(0, NCH // 2, body, 0)
    for b in range(1, NR):
        store_wait(b)


def _make_sc():
    mesh = plsc.VectorSubcoreMesh(core_axis_name="c", subcore_axis_name="s")
    f32 = jnp.float32
    return functools.partial(
        pl.kernel,
        mesh=mesh,
        out_type=jax.ShapeDtypeStruct((B * S, D), f32),
        scratch_types=[pltpu.VMEM((B, S_PER_W), jnp.int32)]
        + [pltpu.VMEM((C, D), f32)] * (2 + NR)
        + [pltpu.SemaphoreType.DMA] * (2 * NR + 2),
    )(_sc_body)


@jax.jit
def kernel(x, tok_table):
    pos = _make_pos()
    out = _make_sc()(pos, x.reshape(B * S), tok_table)
    return out.reshape(B, S, D)


# speedup vs baseline: 2.5698x; 1.0165x over previous
import functools
import math

import jax
import jax.numpy as jnp
from jax import lax
from jax.experimental import pallas as pl
from jax.experimental.pallas import tpu as pltpu
from jax.experimental.pallas import tpu_sc as plsc

VOCAB = 100000
D = 1024
S = 8192
B = 4

NC = 2
NS = 16
NW = NC * NS
S_PER_W = S // NW
C = 16
NCH = S_PER_W // C
NT = NCH * B
NR = 4

NH = S // 64
NL = 64

_LN10000 = math.log(10000.0)


_HPB = 32
_PBS = _HPB * NL


def _inv_denom_cols(shape, dim):
    col = lax.broadcasted_iota(jnp.int32, shape, dim)
    i_val = ((col >> 1) << 1).astype(jnp.float32)
    return jnp.exp(i_val * (_LN10000 / float(D)))


def _pos_body(out_ref, sinl_ref, cosl_ref, a_ref, b_ref):
    g = pl.program_id(0)

    @pl.when(g == 0)
    def _():
        al = lax.broadcasted_iota(jnp.int32, (NL, D), 0).astype(
            jnp.float32
        ) / _inv_denom_cols((NL, D), 1)
        sinl_ref[...] = jnp.sin(al)
        cosl_ref[...] = jnp.cos(al)
        ah = (
            lax.broadcasted_iota(jnp.int32, (NH, D), 0) * 64
        ).astype(jnp.float32) / _inv_denom_cols((NH, D), 1)
        sh = jnp.sin(ah)
        ch_ = jnp.cos(ah)
        col = lax.broadcasted_iota(jnp.int32, (NH, D), 1)
        even = (col & 1) == 0
        a_ref[...] = jnp.where(even, sh, ch_)
        b_ref[...] = jnp.where(even, ch_, -sh)

    hs = pl.ds(g * _HPB, _HPB)
    a = a_ref[hs, :][:, None, :]
    b = b_ref[hs, :][:, None, :]
    sl = sinl_ref[...][None, :, :]
    cl = cosl_ref[...][None, :, :]
    res = a * cl + b * sl
    out_ref[...] = res.reshape(_PBS, D)


def _make_pos():
    return pl.pallas_call(
        _pos_body,
        out_shape=jax.ShapeDtypeStruct((S, D), jnp.float32),
        grid=(S // _PBS,),
        out_specs=pl.BlockSpec((_PBS, D), lambda g: (g, 0)),
        scratch_shapes=[
            pltpu.VMEM((NL, D), jnp.float32),
            pltpu.VMEM((NL, D), jnp.float32),
            pltpu.VMEM((NH, D), jnp.float32),
            pltpu.VMEM((NH, D), jnp.float32),
        ],
    )()


def _add_pos(rows_v, pos_v):
    def add_row(r, _):
        def add_slice(j, _):
            off = j * 16
            rows_v[r, pl.ds(off, 16)] = (
                rows_v[r, pl.ds(off, 16)] + pos_v[r, pl.ds(off, 16)]
            )
            return 0

        lax.fori_loop(0, D // 16, add_slice, 0, unroll=4)
        return 0

    lax.fori_loop(0, C, add_row, 0)


def _sc_body(pos_hbm, x_hbm, tab_hbm, out_hbm, *scr):
    cid = lax.axis_index("c")
    sid = lax.axis_index("s")
    wid = sid * NC + cid
    w0 = wid * S_PER_W

    idx_v = scr[0]
    pos = scr[1:3]
    rows = scr[3 : 3 + NR]
    gsem = scr[3 + NR : 3 + 2 * NR]
    ssem = scr[3 + 2 * NR : 3 + 3 * NR]
    psem = scr[3 + 3 * NR : 3 + 3 * NR + 2]

    def gather_start(ch, b, buf):
        idx = idx_v.at[b, pl.ds(pl.multiple_of(ch * C, C), C)]
        pltpu.make_async_copy(tab_hbm.at[idx], rows[buf], gsem[buf]).start()

    def gather_wait(buf):
        idx = idx_v.at[0, pl.ds(0, C)]
        pltpu.make_async_copy(tab_hbm.at[idx], rows[buf], gsem[buf]).wait()

    def store_start(ch, b, buf):
        base = pl.multiple_of(b * S + w0 + ch * C, C)
        pltpu.make_async_copy(
            rows[buf], out_hbm.at[pl.ds(base, C), :], ssem[buf]
        ).start()

    def store_wait(buf):
        pltpu.make_async_copy(
            rows[buf], out_hbm.at[pl.ds(w0, C), :], ssem[buf]
        ).wait()

    def pos_start(ch, pbuf):
        src = pos_hbm.at[pl.ds(pl.multiple_of(w0 + ch * C, C), C), :]
        pltpu.make_async_copy(src, pos[pbuf], psem[pbuf]).start()

    def pos_wait(pbuf):
        src = pos_hbm.at[pl.ds(pl.multiple_of(w0, C), C), :]
        pltpu.make_async_copy(src, pos[pbuf], psem[pbuf]).wait()

    pos_start(0, 0)
    pos_start(1, 1)
    pltpu.sync_copy(
        x_hbm.at[:, pl.ds(pl.multiple_of(w0, C), S_PER_W)], idx_v
    )
    gather_start(0, 0, 0)

    def body(i, _):
        for half in range(2):
            cc = i * 2 + half
            pbuf = half
            for k in range(B):
                buf = (half * B + k) % NR
                if half * B + k >= NR - 1:
                    store_wait((buf + 1) % NR)
                else:

                    @pl.when(i > 0)
                    def _():
                        store_wait((buf + 1) % NR)

                if k < B - 1:
                    gather_start(cc, k + 1, buf + 1)
                elif half == 0:
                    gather_start(cc + 1, 0, (buf + 1) % NR)
                else:

                    @pl.when(cc < NCH - 1)
                    def _():
                        gather_start(cc + 1, 0, (buf + 1) % NR)

                if k == 0:
                    pos_wait(pbuf)
                gather_wait(buf)
                _add_pos(rows[buf], pos[pbuf])
                store_start(cc, k, buf)
                if k == B - 1:

                    @pl.when(cc + 2 < NCH)
                    def _():
                        pos_start(cc + 2, pbuf)

        return 0

    lax.fori_loop(0, NCH // 2, body, 0)
    for b in range(1, NR):
        store_wait(b)


def _make_sc():
    mesh = plsc.VectorSubcoreMesh(core_axis_name="c", subcore_axis_name="s")
    f32 = jnp.float32
    return functools.partial(
        pl.kernel,
        mesh=mesh,
        out_type=jax.ShapeDtypeStruct((B * S, D), f32),
        scratch_types=[pltpu.VMEM((B, S_PER_W), jnp.int32)]
        + [pltpu.VMEM((C, D), f32)] * (2 + NR)
        + [pltpu.SemaphoreType.DMA] * (2 * NR + 2),
    )(_sc_body)


@jax.jit
def kernel(x, tok_table):
    pos = _make_pos()
    out = _make_sc()(pos, x, tok_table)
    return out.reshape(B, S, D)


# speedup vs baseline: 2.6035x; 1.0131x over previous
import functools
import math

import jax
import jax.numpy as jnp
from jax import lax
from jax.experimental import pallas as pl
from jax.experimental.pallas import tpu as pltpu
from jax.experimental.pallas import tpu_sc as plsc

VOCAB = 100000
D = 1024
S = 8192
B = 4

NC = 2
NS = 16
NW = NC * NS
S_PER_W = S // NW
C = 16
NCH = S_PER_W // C
NT = NCH * B
NR = 4

NH = S // 64
NL = 64

_LN10000 = math.log(10000.0)


_HPB = 32
_PBS = _HPB * NL


def _inv_denom_cols(shape, dim):
    col = lax.broadcasted_iota(jnp.int32, shape, dim)
    i_val = ((col >> 1) << 1).astype(jnp.float32)
    return jnp.exp(i_val * (_LN10000 / float(D)))


def _pos_body(out_ref, sinl_ref, cosl_ref, a_ref, b_ref):
    g = pl.program_id(0)

    @pl.when(g == 0)
    def _():
        al = lax.broadcasted_iota(jnp.int32, (NL, D), 0).astype(
            jnp.float32
        ) / _inv_denom_cols((NL, D), 1)
        sinl_ref[...] = jnp.sin(al)
        cosl_ref[...] = jnp.cos(al)
        ah = (
            lax.broadcasted_iota(jnp.int32, (NH, D), 0) * 64
        ).astype(jnp.float32) / _inv_denom_cols((NH, D), 1)
        sh = jnp.sin(ah)
        ch_ = jnp.cos(ah)
        col = lax.broadcasted_iota(jnp.int32, (NH, D), 1)
        even = (col & 1) == 0
        a_ref[...] = jnp.where(even, sh, ch_)
        b_ref[...] = jnp.where(even, ch_, -sh)

    hs = pl.ds(g * _HPB, _HPB)
    a = a_ref[hs, :][:, None, :]
    b = b_ref[hs, :][:, None, :]
    sl = sinl_ref[...][None, :, :]
    cl = cosl_ref[...][None, :, :]
    res = a * cl + b * sl
    out_ref[...] = res.reshape(_PBS, D)


def _make_pos():
    return pl.pallas_call(
        _pos_body,
        out_shape=jax.ShapeDtypeStruct((S, D), jnp.float32),
        grid=(S // _PBS,),
        out_specs=pl.BlockSpec((_PBS, D), lambda g: (g, 0)),
        scratch_shapes=[
            pltpu.VMEM((NL, D), jnp.float32),
            pltpu.VMEM((NL, D), jnp.float32),
            pltpu.VMEM((NH, D), jnp.float32),
            pltpu.VMEM((NH, D), jnp.float32),
        ],
    )()


def _add_pos(rows_v, pos_v):
    def add_slice(t, _):
        r = t >> 6
        off = (t & 63) * 16
        rows_v[r, pl.ds(off, 16)] = (
            rows_v[r, pl.ds(off, 16)] + pos_v[r, pl.ds(off, 16)]
        )
        return 0

    lax.fori_loop(0, C * (D // 16), add_slice, 0, unroll=8)


def _sc_body(pos_hbm, x_hbm, tab_hbm, out_hbm, *scr):
    cid = lax.axis_index("c")
    sid = lax.axis_index("s")
    wid = sid * NC + cid
    w0 = wid * S_PER_W

    idx_v = scr[0]
    pos = scr[1:3]
    rows = scr[3 : 3 + NR]
    gsem = scr[3 + NR : 3 + 2 * NR]
    ssem = scr[3 + 2 * NR : 3 + 3 * NR]
    psem = scr[3 + 3 * NR : 3 + 3 * NR + 2]

    def gather_start(ch, b, buf):
        idx = idx_v.at[b, pl.ds(pl.multiple_of(ch * C, C), C)]
        pltpu.make_async_copy(tab_hbm.at[idx], rows[buf], gsem[buf]).start()

    def gather_wait(buf):
        idx = idx_v.at[0, pl.ds(0, C)]
        pltpu.make_async_copy(tab_hbm.at[idx], rows[buf], gsem[buf]).wait()

    def store_start(ch, b, buf):
        base = pl.multiple_of(b * S + w0 + ch * C, C)
        pltpu.make_async_copy(
            rows[buf], out_hbm.at[pl.ds(base, C), :], ssem[buf]
        ).start()

    def store_wait(buf):
        pltpu.make_async_copy(
            rows[buf], out_hbm.at[pl.ds(w0, C), :], ssem[buf]
        ).wait()

    def pos_start(ch, pbuf):
        src = pos_hbm.at[pl.ds(pl.multiple_of(w0 + ch * C, C), C), :]
        pltpu.make_async_copy(src, pos[pbuf], psem[pbuf]).start()

    def pos_wait(pbuf):
        src = pos_hbm.at[pl.ds(pl.multiple_of(w0, C), C), :]
        pltpu.make_async_copy(src, pos[pbuf], psem[pbuf]).wait()

    pos_start(0, 0)
    pos_start(1, 1)
    pltpu.sync_copy(
        x_hbm.at[:, pl.ds(pl.multiple_of(w0, C), S_PER_W)], idx_v
    )
    gather_start(0, 0, 0)

    def body(i, _):
        for half in range(2):
            cc = i * 2 + half
            pbuf = half
            for k in range(B):
                buf = (half * B + k) % NR
                if half * B + k >= NR - 1:
                    store_wait((buf + 1) % NR)
                else:

                    @pl.when(i > 0)
                    def _():
                        store_wait((buf + 1) % NR)

                if k < B - 1:
                    gather_start(cc, k + 1, buf + 1)
                elif half == 0:
                    gather_start(cc + 1, 0, (buf + 1) % NR)
                else:

                    @pl.when(cc < NCH - 1)
                    def _():
                        gather_start(cc + 1, 0, (buf + 1) % NR)

                if k == 0:
                    pos_wait(pbuf)
                gather_wait(buf)
                _add_pos(rows[buf], pos[pbuf])
                store_start(cc, k, buf)
                if k == B - 1:

                    @pl.when(cc + 2 < NCH)
                    def _():
                        pos_start(cc + 2, pbuf)

        return 0

    lax.fori_loop(0, NCH // 2, body, 0)
    for b in range(1, NR):
        store_wait(b)


def _make_sc():
    mesh = plsc.VectorSubcoreMesh(core_axis_name="c", subcore_axis_name="s")
    f32 = jnp.float32
    return functools.partial(
        pl.kernel,
        mesh=mesh,
        out_type=jax.ShapeDtypeStruct((B * S, D), f32),
        scratch_types=[pltpu.VMEM((B, S_PER_W), jnp.int32)]
        + [pltpu.VMEM((C, D), f32)] * (2 + NR)
        + [pltpu.SemaphoreType.DMA] * (2 * NR + 2),
    )(_sc_body)


@jax.jit
def kernel(x, tok_table):
    pos = _make_pos()
    out = _make_sc()(pos, x, tok_table)
    return out.reshape(B, S, D)
